# Initial kernel scaffold; baseline (speedup 1.0000x reference)
import jax
import jax.numpy as jnp
from jax.experimental import pallas as pl


def kernel(nfeats, edge_index, efeats, W_msg, b_msg, W_apply, b_apply):
    raise NotImplementedError("write your pallas kernel here")


# speedup vs baseline: 2.6165x; 2.6165x over previous
import functools

import jax
import jax.numpy as jnp
from jax import lax
from jax.experimental import pallas as pl
from jax.experimental.pallas import tpu as pltpu
from jax.experimental.pallas import tpu_sc as plsc

NC = 2
NS = 16
K = 80


def _rewrite_dst(dst_v, c, dloc_v, nbase, half):
    for j in range(K // 16):
        d = dst_v[c, pl.ds(j * 16, 16)]
        loc = d - nbase
        oob = (loc < 0) | (loc >= half)
        dloc_v[pl.ds(j * 16, 16)] = jnp.where(oob, half, loc)


def _zsplit(hpad):
    zrows = hpad // NS // 8 * 8
    return zrows, NS * zrows, hpad - NS * zrows


def _make_sc_xsum(n, e, din):
    nchunk = e // NS // K
    half = n // NC
    hpad = half + 8
    zrows, ztail0, ztail = _zsplit(hpad)
    mesh = plsc.VectorSubcoreMesh(core_axis_name="c", subcore_axis_name="s")

    @functools.partial(
        pl.kernel,
        out_type=jax.ShapeDtypeStruct((NC, hpad, din), jnp.float32),
        mesh=mesh,
        scratch_types=[
            pltpu.VMEM((nchunk, K), jnp.int32),
            pltpu.VMEM((nchunk, K), jnp.int32),
            pltpu.VMEM((K,), jnp.int32),
            pltpu.VMEM((K, din), jnp.float32),
            pltpu.VMEM_SHARED((hpad, din), jnp.float32),
            pltpu.SemaphoreType.DMA,
        ],
    )
    def sc_fn(nf_hbm, srcr_hbm, dstr_hbm, zx_hbm, sx_out,
              src_v, dst_v, dloc_v, rows_v, sx_sh, sem):
        cid = lax.axis_index("c")
        sid = lax.axis_index("s")

        r0 = sid * zrows
        pltpu.sync_copy(zx_hbm.at[pl.ds(r0, zrows)], sx_sh.at[pl.ds(r0, zrows)])

        @pl.when(sid == 0)
        def _zero_tail():
            pltpu.sync_copy(zx_hbm.at[pl.ds(ztail0, ztail)],
                            sx_sh.at[pl.ds(ztail0, ztail)])

        pltpu.sync_copy(srcr_hbm.at[sid], src_v)
        pltpu.sync_copy(dstr_hbm.at[sid], dst_v)
        plsc.subcore_barrier()

        nbase = cid * half

        def body(c, carry):
            gather = pltpu.async_copy(nf_hbm.at[src_v.at[c]], rows_v, sem)
            _rewrite_dst(dst_v, c, dloc_v, nbase, half)
            gather.wait()
            pltpu.sync_copy(rows_v, sx_sh.at[dloc_v], add=True)
            return carry

        lax.fori_loop(0, nchunk, body, 0)
        plsc.subcore_barrier()

        pltpu.sync_copy(sx_sh.at[pl.ds(r0, zrows)],
                        sx_out.at[cid, pl.ds(r0, zrows)])

        @pl.when(sid == 0)
        def _write_tail():
            pltpu.sync_copy(sx_sh.at[pl.ds(ztail0, ztail)],
                            sx_out.at[cid, pl.ds(ztail0, ztail)])

    return sc_fn


def _make_sc_efsum(n, e, de):
    nchunk = e // NS // K
    half = n // NC
    hpad = half + 8
    zrows, ztail0, ztail = _zsplit(hpad)
    mesh = plsc.VectorSubcoreMesh(core_axis_name="c", subcore_axis_name="s")

    @functools.partial(
        pl.kernel,
        out_type=jax.ShapeDtypeStruct((NC, hpad, 128), jnp.float32),
        mesh=mesh,
        scratch_types=[
            pltpu.VMEM((nchunk, K), jnp.int32),
            pltpu.VMEM((K,), jnp.int32),
            pltpu.VMEM((K, de), jnp.float32),
            pltpu.VMEM((K, 128), jnp.float32),
            pltpu.VMEM_SHARED((hpad, 128), jnp.float32),
            pltpu.SemaphoreType.DMA,
        ],
    )
    def sc_fn(dstr_hbm, ef_hbm, padc_hbm, zx_hbm, sef_out,
              dst_v, dloc_v, ef_v, pad_v, sef_sh, sem):
        cid = lax.axis_index("c")
        sid = lax.axis_index("s")

        r0 = sid * zrows
        pltpu.sync_copy(zx_hbm.at[pl.ds(r0, zrows)],
                        sef_sh.at[pl.ds(r0, zrows)])

        @pl.when(sid == 0)
        def _zero_tail():
            pltpu.sync_copy(zx_hbm.at[pl.ds(ztail0, ztail)],
                            sef_sh.at[pl.ds(ztail0, ztail)])

        pltpu.sync_copy(dstr_hbm.at[sid], dst_v)
        pltpu.sync_copy(padc_hbm, pad_v)
        plsc.subcore_barrier()

        ebase = sid * (nchunk * K)
        nbase = cid * half

        def body(c, carry):
            pltpu.sync_copy(ef_hbm.at[pl.ds(ebase + c * K, K)], ef_v)
            _rewrite_dst(dst_v, c, dloc_v, nbase, half)

            def cp(i, carry2):
                pad_v[i, pl.ds(0, de)] = ef_v[i, pl.ds(0, de)]
                return carry2

            lax.fori_loop(0, K, cp, 0)
            pltpu.sync_copy(pad_v, sef_sh.at[dloc_v], add=True)
            return carry

        lax.fori_loop(0, nchunk, body, 0)
        plsc.subcore_barrier()

        pltpu.sync_copy(sef_sh.at[pl.ds(r0, zrows)],
                        sef_out.at[cid, pl.ds(r0, zrows)])

        @pl.when(sid == 0)
        def _write_tail():
            pltpu.sync_copy(sef_sh.at[pl.ds(ztail0, ztail)],
                            sef_out.at[cid, pl.ds(ztail0, ztail)])

    return sc_fn


def _tc_dense(nf_ref, sx_ref, sef_ref, wmx_ref, wme_ref, wax_ref,
              wah_ref, bm_ref, ba_ref, out_ref):
    sef = sef_ref[0]
    se = sef[:, :16]
    cnt = sef[:, 16:17]
    sums = (
        jnp.dot(sx_ref[0], wmx_ref[...], preferred_element_type=jnp.float32,
                precision=lax.Precision.HIGHEST)
        + jnp.dot(se, wme_ref[...], preferred_element_type=jnp.float32,
                  precision=lax.Precision.HIGHEST)
        + cnt * bm_ref[...]
    )
    h_neigh = sums / jnp.maximum(cnt, 1.0)
    h = (
        jnp.dot(nf_ref[...], wax_ref[...], preferred_element_type=jnp.float32,
                precision=lax.Precision.HIGHEST)
        + jnp.dot(h_neigh, wah_ref[...], preferred_element_type=jnp.float32,
                  precision=lax.Precision.HIGHEST)
        + ba_ref[...]
    )
    out_ref[...] = jnp.maximum(h, 0.0)


def kernel(nfeats, edge_index, efeats, W_msg, b_msg, W_apply, b_apply):
    n, _, din = nfeats.shape
    e = edge_index.shape[1]
    de = efeats.shape[2]
    dout = W_msg.shape[0]
    half = n // NC
    hpad = half + 8

    nf2 = nfeats.reshape(n, din)
    src_r = edge_index[0].reshape(NS, e // NS // K, K)
    dst_r = edge_index[1].reshape(NS, e // NS // K, K)
    ef2 = efeats.reshape(e, de)
    padc = jnp.concatenate(
        [jnp.zeros((K, de), jnp.float32), jnp.ones((K, 1), jnp.float32),
         jnp.zeros((K, 128 - de - 1), jnp.float32)], axis=1)
    zx = jnp.zeros((hpad, din), jnp.float32)

    sx_sum = _make_sc_xsum(n, e, din)(nf2, src_r, dst_r, zx)
    sef_sum = _make_sc_efsum(n, e, de)(dst_r, ef2, padc, zx)

    blk = 1000
    bph = half // blk
    grid = n // blk

    def _half_map(i):
        return (i // bph, i % bph, 0)

    out = pl.pallas_call(
        _tc_dense,
        grid=(grid,),
        in_specs=[
            pl.BlockSpec((blk, din), lambda i: (i, 0)),
            pl.BlockSpec((1, blk, din), _half_map),
            pl.BlockSpec((1, blk, 128), _half_map),
            pl.BlockSpec((din, dout), lambda i: (0, 0)),
            pl.BlockSpec((de, dout), lambda i: (0, 0)),
            pl.BlockSpec((din, dout), lambda i: (0, 0)),
            pl.BlockSpec((dout, dout), lambda i: (0, 0)),
            pl.BlockSpec((1, dout), lambda i: (0, 0)),
            pl.BlockSpec((1, dout), lambda i: (0, 0)),
        ],
        out_specs=pl.BlockSpec((blk, dout), lambda i: (i, 0)),
        out_shape=jax.ShapeDtypeStruct((n, dout), jnp.float32),
    )(
        nf2, sx_sum, sef_sum,
        W_msg[:, :din].T, W_msg[:, din:].T,
        W_apply[:, :din].T, W_apply[:, din:].T,
        b_msg.reshape(1, dout), b_apply.reshape(1, dout),
    )
    return out.reshape(n, 1, dout)


# speedup vs baseline: 3.8168x; 1.4587x over previous
import functools

import jax
import jax.numpy as jnp
from jax import lax
from jax.experimental import pallas as pl
from jax.experimental.pallas import tpu as pltpu
from jax.experimental.pallas import tpu_sc as plsc

NC = 2
NS = 16
K = 80
NB = 5
NBA = 2


def _rewrite_dst(dst_v, c, dloc_v, nbase, half):
    for j in range(K // 16):
        d = dst_v[c, pl.ds(j * 16, 16)]
        loc = d - nbase
        oob = (loc < 0) | (loc >= half)
        dloc_v[pl.ds(j * 16, 16)] = jnp.where(oob, half, loc)


def _zsplit(hpad):
    zrows = hpad // NS // 8 * 8
    return zrows, NS * zrows, hpad - NS * zrows


def _make_sc_xsum(n, e, din):
    nchunk = e // NS // K
    nbody = nchunk // NBA
    half = n // NC
    hpad = half + 8
    zrows, ztail0, ztail = _zsplit(hpad)
    mesh = plsc.VectorSubcoreMesh(core_axis_name="c", subcore_axis_name="s")

    @functools.partial(
        pl.kernel,
        out_type=jax.ShapeDtypeStruct((NC, hpad, din), jnp.float32),
        mesh=mesh,
        scratch_types=[
            pltpu.VMEM((nchunk, K), jnp.int32),
            pltpu.VMEM((nchunk, K), jnp.int32),
            [pltpu.VMEM((K,), jnp.int32) for _ in range(NBA)],
            [pltpu.VMEM((K, din), jnp.float32) for _ in range(NBA)],
            pltpu.VMEM_SHARED((hpad, din), jnp.float32),
            pltpu.SemaphoreType.DMA,
            pltpu.SemaphoreType.DMA,
        ],
    )
    def sc_fn(nf_hbm, srcr_hbm, dstr_hbm, zx_hbm, sx_out,
              src_v, dst_v, dlocs, rows, sx_sh, sem_g, sem_s):
        cid = lax.axis_index("c")
        sid = lax.axis_index("s")

        r0 = sid * zrows
        pltpu.sync_copy(zx_hbm.at[pl.ds(r0, zrows)], sx_sh.at[pl.ds(r0, zrows)])

        @pl.when(sid == 0)
        def _zero_tail():
            pltpu.sync_copy(zx_hbm.at[pl.ds(ztail0, ztail)],
                            sx_sh.at[pl.ds(ztail0, ztail)])

        pltpu.sync_copy(srcr_hbm.at[sid], src_v)
        pltpu.sync_copy(dstr_hbm.at[sid], dst_v)
        plsc.subcore_barrier()

        nbase = cid * half

        def body(b, carry):
            c0 = b * NBA
            gs = [pltpu.async_copy(nf_hbm.at[src_v.at[c0 + t]], rows[t], sem_g)
                  for t in range(NBA)]
            ss = []
            for t in range(NBA):
                gs[t].wait()
                _rewrite_dst(dst_v, c0 + t, dlocs[t], nbase, half)
                ss.append(pltpu.async_copy(rows[t], sx_sh.at[dlocs[t]], sem_s,
                                           add=True))
            for t in range(NBA):
                ss[t].wait()
            return carry

        lax.fori_loop(0, nbody, body, 0)
        plsc.subcore_barrier()

        pltpu.sync_copy(sx_sh.at[pl.ds(r0, zrows)],
                        sx_out.at[cid, pl.ds(r0, zrows)])

        @pl.when(sid == 0)
        def _write_tail():
            pltpu.sync_copy(sx_sh.at[pl.ds(ztail0, ztail)],
                            sx_out.at[cid, pl.ds(ztail0, ztail)])

    return sc_fn


def _make_sc_efsum(n, e, de):
    nchunk = e // NS // K
    nbody = nchunk // NB
    half = n // NC
    hpad = half + 8
    zrows, ztail0, ztail = _zsplit(hpad)
    mesh = plsc.VectorSubcoreMesh(core_axis_name="c", subcore_axis_name="s")

    @functools.partial(
        pl.kernel,
        out_type=jax.ShapeDtypeStruct((NC, hpad, 128), jnp.float32),
        mesh=mesh,
        scratch_types=[
            pltpu.VMEM((nchunk, K), jnp.int32),
            [pltpu.VMEM((K,), jnp.int32) for _ in range(NB)],
            [pltpu.VMEM((K * de,), jnp.float32) for _ in range(NB)],
            [pltpu.VMEM((K, 128), jnp.float32) for _ in range(NB)],
            pltpu.VMEM_SHARED((hpad, 128), jnp.float32),
            pltpu.SemaphoreType.DMA,
            pltpu.SemaphoreType.DMA,
        ],
    )
    def sc_fn(dstr_hbm, ef_hbm, padc_hbm, zx_hbm, sef_out,
              dst_v, dlocs, efs, pads, sef_sh, sem_g, sem_s):
        cid = lax.axis_index("c")
        sid = lax.axis_index("s")

        r0 = sid * zrows
        pltpu.sync_copy(zx_hbm.at[pl.ds(r0, zrows)],
                        sef_sh.at[pl.ds(r0, zrows)])

        @pl.when(sid == 0)
        def _zero_tail():
            pltpu.sync_copy(zx_hbm.at[pl.ds(ztail0, ztail)],
                            sef_sh.at[pl.ds(ztail0, ztail)])

        pltpu.sync_copy(dstr_hbm.at[sid], dst_v)
        for t in range(NB):
            pltpu.sync_copy(padc_hbm, pads[t])
        plsc.subcore_barrier()

        ebase = sid * (nchunk * K * de)
        nbase = cid * half

        def body(b, carry):
            c0 = b * NB
            gs = [pltpu.async_copy(
                      ef_hbm.at[pl.ds(ebase + (c0 + t) * (K * de), K * de)],
                      efs[t], sem_g)
                  for t in range(NB)]
            ss = []
            for t in range(NB):
                gs[t].wait()
                _rewrite_dst(dst_v, c0 + t, dlocs[t], nbase, half)
                for i in range(K):
                    pads[t][i, pl.ds(0, de)] = efs[t][pl.ds(i * de, de)]
                ss.append(pltpu.async_copy(pads[t], sef_sh.at[dlocs[t]], sem_s,
                                           add=True))
            for t in range(NB):
                ss[t].wait()
            return carry

        lax.fori_loop(0, nbody, body, 0)
        plsc.subcore_barrier()

        pltpu.sync_copy(sef_sh.at[pl.ds(r0, zrows)],
                        sef_out.at[cid, pl.ds(r0, zrows)])

        @pl.when(sid == 0)
        def _write_tail():
            pltpu.sync_copy(sef_sh.at[pl.ds(ztail0, ztail)],
                            sef_out.at[cid, pl.ds(ztail0, ztail)])

    return sc_fn


def _tc_dense(nf_ref, sx_ref, sef_ref, wmx_ref, wme_ref, wax_ref,
              wah_ref, bm_ref, ba_ref, out_ref):
    sef = sef_ref[0]
    se = sef[:, :16]
    cnt = sef[:, 16:17]
    sums = (
        jnp.dot(sx_ref[0], wmx_ref[...], preferred_element_type=jnp.float32,
                precision=lax.Precision.HIGHEST)
        + jnp.dot(se, wme_ref[...], preferred_element_type=jnp.float32,
                  precision=lax.Precision.HIGHEST)
        + cnt * bm_ref[...]
    )
    h_neigh = sums / jnp.maximum(cnt, 1.0)
    h = (
        jnp.dot(nf_ref[...], wax_ref[...], preferred_element_type=jnp.float32,
                precision=lax.Precision.HIGHEST)
        + jnp.dot(h_neigh, wah_ref[...], preferred_element_type=jnp.float32,
                  precision=lax.Precision.HIGHEST)
        + ba_ref[...]
    )
    out_ref[...] = jnp.maximum(h, 0.0)


def kernel(nfeats, edge_index, efeats, W_msg, b_msg, W_apply, b_apply):
    n, _, din = nfeats.shape
    e = edge_index.shape[1]
    de = efeats.shape[2]
    dout = W_msg.shape[0]
    half = n // NC
    hpad = half + 8

    nf2 = nfeats.reshape(n, din)
    src_r = edge_index[0].reshape(NS, e // NS // K, K)
    dst_r = edge_index[1].reshape(NS, e // NS // K, K)
    ef_flat = efeats.reshape(e * de)
    padc = jnp.concatenate(
        [jnp.zeros((K, de), jnp.float32), jnp.ones((K, 1), jnp.float32),
         jnp.zeros((K, 128 - de - 1), jnp.float32)], axis=1)
    zx = jnp.zeros((hpad, din), jnp.float32)

    sx_sum = _make_sc_xsum(n, e, din)(nf2, src_r, dst_r, zx)
    zx_b = zx + sx_sum[0, 0, 0] * 0.0
    sef_sum = _make_sc_efsum(n, e, de)(dst_r, ef_flat, padc, zx_b)

    blk = 1000
    bph = half // blk
    grid = n // blk

    def _half_map(i):
        return (i // bph, i % bph, 0)

    out = pl.pallas_call(
        _tc_dense,
        grid=(grid,),
        in_specs=[
            pl.BlockSpec((blk, din), lambda i: (i, 0)),
            pl.BlockSpec((1, blk, din), _half_map),
            pl.BlockSpec((1, blk, 128), _half_map),
            pl.BlockSpec((din, dout), lambda i: (0, 0)),
            pl.BlockSpec((de, dout), lambda i: (0, 0)),
            pl.BlockSpec((din, dout), lambda i: (0, 0)),
            pl.BlockSpec((dout, dout), lambda i: (0, 0)),
            pl.BlockSpec((1, dout), lambda i: (0, 0)),
            pl.BlockSpec((1, dout), lambda i: (0, 0)),
        ],
        out_specs=pl.BlockSpec((blk, dout), lambda i: (i, 0)),
        out_shape=jax.ShapeDtypeStruct((n, dout), jnp.float32),
    )(
        nf2, sx_sum, sef_sum,
        W_msg[:, :din].T, W_msg[:, din:].T,
        W_apply[:, :din].T, W_apply[:, din:].T,
        b_msg.reshape(1, dout), b_apply.reshape(1, dout),
    )
    return out.reshape(n, 1, dout)


# speedup vs baseline: 4.5276x; 1.1862x over previous
import functools

import jax
import jax.numpy as jnp
from jax import lax
from jax.experimental import pallas as pl
from jax.experimental.pallas import tpu as pltpu
from jax.experimental.pallas import tpu_sc as plsc

NC = 2
NS = 16
K = 80
NB = 5
NBA = 2
NTRASH = 8


def _rewrite_dst(dst_v, c, dloc_v, nbase, half):
    for j in range(K // 16):
        d = dst_v[c, pl.ds(j * 16, 16)]
        loc = d - nbase
        oob = (loc < 0) | (loc >= half)
        dloc_v[pl.ds(j * 16, 16)] = jnp.where(oob, half + (d & (NTRASH - 1)),
                                              loc)


def _zsplit(hpad):
    zrows = hpad // NS // 8 * 8
    return zrows, NS * zrows, hpad - NS * zrows


def _make_sc_xsum(n, e, din):
    nchunk = e // NS // K
    nbody = nchunk // NBA
    half = n // NC
    hpad = half + NTRASH
    zrows, ztail0, ztail = _zsplit(hpad)
    mesh = plsc.VectorSubcoreMesh(core_axis_name="c", subcore_axis_name="s")

    @functools.partial(
        pl.kernel,
        out_type=jax.ShapeDtypeStruct((NC, hpad, din), jnp.float32),
        mesh=mesh,
        scratch_types=[
            pltpu.VMEM((nchunk, K), jnp.int32),
            pltpu.VMEM((nchunk, K), jnp.int32),
            [pltpu.VMEM((K,), jnp.int32) for _ in range(NBA)],
            [pltpu.VMEM((K, din), jnp.float32) for _ in range(NBA)],
            pltpu.VMEM_SHARED((hpad, din), jnp.float32),
            pltpu.SemaphoreType.DMA,
            pltpu.SemaphoreType.DMA,
        ],
    )
    def sc_fn(nf_hbm, srcr_hbm, dstr_hbm, zx_hbm, sx_out,
              src_v, dst_v, dlocs, rows, sx_sh, sem_g, sem_s):
        cid = lax.axis_index("c")
        sid = lax.axis_index("s")

        r0 = sid * zrows
        pltpu.sync_copy(zx_hbm.at[pl.ds(r0, zrows)], sx_sh.at[pl.ds(r0, zrows)])

        @pl.when(sid == 0)
        def _zero_tail():
            pltpu.sync_copy(zx_hbm.at[pl.ds(ztail0, ztail)],
                            sx_sh.at[pl.ds(ztail0, ztail)])

        pltpu.sync_copy(srcr_hbm.at[sid], src_v)
        pltpu.sync_copy(dstr_hbm.at[sid], dst_v)
        plsc.subcore_barrier()

        nbase = cid * half

        def group(c0, width):
            gs = [pltpu.async_copy(nf_hbm.at[src_v.at[c0 + t]], rows[t], sem_g)
                  for t in range(width)]
            ss = []
            for t in range(width):
                gs[t].wait()
                _rewrite_dst(dst_v, c0 + t, dlocs[t], nbase, half)
                ss.append(pltpu.async_copy(rows[t], sx_sh.at[dlocs[t]], sem_s,
                                           add=True))
            for t in range(width):
                ss[t].wait()

        def body(b, carry):
            group(b * NBA, NBA)
            return carry

        lax.fori_loop(0, nbody, body, 0)
        if nchunk % NBA:
            group(nbody * NBA, nchunk % NBA)
        plsc.subcore_barrier()

        pltpu.sync_copy(sx_sh.at[pl.ds(r0, zrows)],
                        sx_out.at[cid, pl.ds(r0, zrows)])

        @pl.when(sid == 0)
        def _write_tail():
            pltpu.sync_copy(sx_sh.at[pl.ds(ztail0, ztail)],
                            sx_out.at[cid, pl.ds(ztail0, ztail)])

    return sc_fn


def _make_sc_efsum(n, e, de):
    nchunk = e // NS // K
    nbody = nchunk // NB
    half = n // NC
    hpad = half + NTRASH
    zrows, ztail0, ztail = _zsplit(hpad)
    mesh = plsc.VectorSubcoreMesh(core_axis_name="c", subcore_axis_name="s")

    @functools.partial(
        pl.kernel,
        out_type=jax.ShapeDtypeStruct((NC, hpad, 128), jnp.float32),
        mesh=mesh,
        scratch_types=[
            pltpu.VMEM((nchunk, K), jnp.int32),
            [pltpu.VMEM((K,), jnp.int32) for _ in range(NB)],
            [pltpu.VMEM((K * de,), jnp.float32) for _ in range(NB)],
            [pltpu.VMEM((K, 128), jnp.float32) for _ in range(NB)],
            pltpu.VMEM_SHARED((hpad, 128), jnp.float32),
            pltpu.SemaphoreType.DMA,
            pltpu.SemaphoreType.DMA,
        ],
    )
    def sc_fn(dstr_hbm, ef_hbm, padc_hbm, zx_hbm, sef_out,
              dst_v, dlocs, efs, pads, sef_sh, sem_g, sem_s):
        cid = lax.axis_index("c")
        sid = lax.axis_index("s")

        r0 = sid * zrows
        pltpu.sync_copy(zx_hbm.at[pl.ds(r0, zrows)],
                        sef_sh.at[pl.ds(r0, zrows)])

        @pl.when(sid == 0)
        def _zero_tail():
            pltpu.sync_copy(zx_hbm.at[pl.ds(ztail0, ztail)],
                            sef_sh.at[pl.ds(ztail0, ztail)])

        pltpu.sync_copy(dstr_hbm.at[sid], dst_v)
        for t in range(NB):
            pltpu.sync_copy(padc_hbm, pads[t])
        plsc.subcore_barrier()

        ebase = sid * (nchunk * K * de)
        nbase = cid * half

        def body(b, carry):
            c0 = b * NB
            gs = [pltpu.async_copy(
                      ef_hbm.at[pl.ds(ebase + (c0 + t) * (K * de), K * de)],
                      efs[t], sem_g)
                  for t in range(NB)]
            ss = []
            for t in range(NB):
                gs[t].wait()
                _rewrite_dst(dst_v, c0 + t, dlocs[t], nbase, half)
                for i in range(K):
                    pads[t][i, pl.ds(0, de)] = efs[t][pl.ds(i * de, de)]
                ss.append(pltpu.async_copy(pads[t], sef_sh.at[dlocs[t]], sem_s,
                                           add=True))
            for t in range(NB):
                ss[t].wait()
            return carry

        lax.fori_loop(0, nbody, body, 0)
        plsc.subcore_barrier()

        pltpu.sync_copy(sef_sh.at[pl.ds(r0, zrows)],
                        sef_out.at[cid, pl.ds(r0, zrows)])

        @pl.when(sid == 0)
        def _write_tail():
            pltpu.sync_copy(sef_sh.at[pl.ds(ztail0, ztail)],
                            sef_out.at[cid, pl.ds(ztail0, ztail)])

    return sc_fn


def _tc_dense(nf_ref, sx_ref, sef_ref, wmx_ref, wme_ref, wax_ref,
              wah_ref, bm_ref, ba_ref, out_ref):
    sef = sef_ref[0]
    se = sef[:, :16]
    cnt = sef[:, 16:17]
    sums = (
        jnp.dot(sx_ref[0], wmx_ref[...], preferred_element_type=jnp.float32,
                precision=lax.Precision.HIGHEST)
        + jnp.dot(se, wme_ref[...], preferred_element_type=jnp.float32,
                  precision=lax.Precision.HIGHEST)
        + cnt * bm_ref[...]
    )
    h_neigh = sums / jnp.maximum(cnt, 1.0)
    h = (
        jnp.dot(nf_ref[...], wax_ref[...], preferred_element_type=jnp.float32,
                precision=lax.Precision.HIGHEST)
        + jnp.dot(h_neigh, wah_ref[...], preferred_element_type=jnp.float32,
                  precision=lax.Precision.HIGHEST)
        + ba_ref[...]
    )
    out_ref[...] = jnp.maximum(h, 0.0)


def kernel(nfeats, edge_index, efeats, W_msg, b_msg, W_apply, b_apply):
    n, _, din = nfeats.shape
    e = edge_index.shape[1]
    de = efeats.shape[2]
    dout = W_msg.shape[0]
    half = n // NC
    hpad = half + NTRASH

    nf2 = nfeats.reshape(n, din)
    src_r = edge_index[0].reshape(NS, e // NS // K, K)
    dst_r = edge_index[1].reshape(NS, e // NS // K, K)
    ef_flat = efeats.reshape(e * de)
    padc = jnp.concatenate(
        [jnp.zeros((K, de), jnp.float32), jnp.ones((K, 1), jnp.float32),
         jnp.zeros((K, 128 - de - 1), jnp.float32)], axis=1)
    zx = jnp.zeros((hpad, din), jnp.float32)

    sx_sum = _make_sc_xsum(n, e, din)(nf2, src_r, dst_r, zx)
    zx_b = zx + sx_sum[0, 0, 0] * 0.0
    sef_sum = _make_sc_efsum(n, e, de)(dst_r, ef_flat, padc, zx_b)

    blk = 1000
    bph = half // blk
    grid = n // blk

    def _half_map(i):
        return (i // bph, i % bph, 0)

    out = pl.pallas_call(
        _tc_dense,
        grid=(grid,),
        in_specs=[
            pl.BlockSpec((blk, din), lambda i: (i, 0)),
            pl.BlockSpec((1, blk, din), _half_map),
            pl.BlockSpec((1, blk, 128), _half_map),
            pl.BlockSpec((din, dout), lambda i: (0, 0)),
            pl.BlockSpec((de, dout), lambda i: (0, 0)),
            pl.BlockSpec((din, dout), lambda i: (0, 0)),
            pl.BlockSpec((dout, dout), lambda i: (0, 0)),
            pl.BlockSpec((1, dout), lambda i: (0, 0)),
            pl.BlockSpec((1, dout), lambda i: (0, 0)),
        ],
        out_specs=pl.BlockSpec((blk, dout), lambda i: (i, 0)),
        out_shape=jax.ShapeDtypeStruct((n, dout), jnp.float32),
    )(
        nf2, sx_sum, sef_sum,
        W_msg[:, :din].T, W_msg[:, din:].T,
        W_apply[:, :din].T, W_apply[:, din:].T,
        b_msg.reshape(1, dout), b_apply.reshape(1, dout),
    )
    return out.reshape(n, 1, dout)
